# baseline (speedup 1.0000x reference)
import functools

import jax
import jax.numpy as jnp
import numpy as np
from jax import lax
from jax.experimental import pallas as pl
from jax.experimental.pallas import tpu as pltpu
from jax.experimental.pallas import tpu_sc as plsc

MAX_RADIUS = 5.0
AVG_DEG = 32.0
NUM_BASIS = 16

NC = 2
NS = 16
LANES = 16

_COS_PI = (
    9.9999999989e-01, -4.9348021859e+00, 4.0587118172e+00, -1.3352602858e+00,
    2.3532082435e-01, -2.5785806878e-02, 1.9043274683e-03, -8.8690476959e-05,
)


def _coeff_body(r_ref, w1t_ref, b1_ref, w2t_ref, b2_ref, w3_ref, b3_ref,
                out_ref):
    r = r_ref[0]
    inv_step = np.float32((NUM_BASIS - 1) / MAX_RADIUS)
    k_col = lax.broadcasted_iota(
        jnp.int32, (NUM_BASIS, 1), 0).astype(jnp.float32)
    x = r * inv_step - k_col
    u = x * x
    p = jnp.full_like(u, np.float32(_COS_PI[-1]))
    for coef in _COS_PI[-2::-1]:
        p = p * u + np.float32(coef)
    basis = jnp.where(u < 1.0, 0.5 + 0.5 * p, 0.0)
    h = jnp.dot(w1t_ref[...], basis, preferred_element_type=jnp.float32)
    h = jnp.maximum(h + b1_ref[...], 0.0)
    h = jnp.dot(w2t_ref[...], h, preferred_element_type=jnp.float32)
    h = jnp.maximum(h + b2_ref[...], 0.0)
    out = lax.dot_general(h, w3_ref[...], (((0,), (0,)), ((), ())),
                          preferred_element_type=jnp.float32)
    out_ref[...] = (out + b3_ref[...]).astype(jnp.bfloat16)


def _coeff_call(radii2, W1t, b1, W2t, b2, W3, b3, block_e):
    grid, _, BE = radii2.shape
    H, NB = W1t.shape
    D = W3.shape[1]
    assert BE == block_e
    full = lambda shape: pl.BlockSpec(shape, lambda i: (0, 0))
    return pl.pallas_call(
        _coeff_body,
        grid=(grid,),
        in_specs=[
            pl.BlockSpec((1, 1, block_e), lambda i: (i, 0, 0)),
            full((H, NB)),
            full((H, 1)),
            full((H, H)),
            full((H, 1)),
            full((H, D)),
            full((1, D)),
        ],
        out_specs=pl.BlockSpec((block_e, D), lambda i: (i, 0)),
        out_shape=jax.ShapeDtypeStruct((grid * block_e, D), jnp.bfloat16),
    )(radii2, W1t, b1, W2t, b2, W3, b3)


def _prep_body(x_ref, p_ref, out_ref):
    out_ref[...] = jnp.dot(x_ref[...], p_ref[...],
                           preferred_element_type=jnp.float32
                           ).astype(jnp.bfloat16)


def _prep_call(features, P, block_n):
    N, D = features.shape
    return pl.pallas_call(
        _prep_body,
        grid=(N // block_n,),
        in_specs=[pl.BlockSpec((block_n, D), lambda i: (i, 0)),
                  pl.BlockSpec((D, D), lambda i: (0, 0))],
        out_specs=pl.BlockSpec((block_n, D), lambda i: (i, 0)),
        out_shape=jax.ShapeDtypeStruct((N, D), jnp.bfloat16),
    )(features, P)


def _sc_edge_body(N_pad, D, C, n_chunks,
                  feat_hbm, coeff_hbm, src_hbm, dst_hbm, out_hbm,
                  src_b0, src_b1, dst_b0, dst_b1, rows_v, coeff_v, msg_v,
                  acc_sh, g0, g1, c0, c1, is0, is1, id0, id1, s_sem):
    NBUF = 2
    src_b = (src_b0, src_b1)
    dst_b = (dst_b0, dst_b1)
    g_sems = (g0, g1)
    c_sems = (c0, c1)
    is_sems = (is0, is1)
    id_sems = (id0, id1)
    c = lax.axis_index("c")
    s = lax.axis_index("s")
    wid = s * NC + c
    rpt = N_pad // NS

    def zrow(j, carry):
        for k in range(D // LANES):
            msg_v[j, pl.ds(k * LANES, LANES)] = jnp.zeros((LANES,),
                                                          jnp.float32)
        return carry

    lax.fori_loop(0, C, zrow, 0)
    for t in range(rpt // C):
        pltpu.sync_copy(msg_v, acc_sh.at[pl.ds(s * rpt + t * C, C), :])

    def idx_load(j, b):
        pltpu.async_copy(src_hbm.at[wid, j], src_b[b], is_sems[b])

    def idx_wait(j, b):
        pltpu.make_async_copy(src_hbm.at[wid, j], src_b[b], is_sems[b]).wait()

    def dst_load(j, b):
        pltpu.async_copy(dst_hbm.at[wid, j], dst_b[b], id_sems[b])

    def dst_wait(j, b):
        pltpu.make_async_copy(dst_hbm.at[wid, j], dst_b[b], id_sems[b]).wait()

    def load(j, b):
        pltpu.async_copy(feat_hbm.at[src_b[b]], rows_v.at[b], g_sems[b])
        pltpu.async_copy(coeff_hbm.at[wid, j], coeff_v.at[b], c_sems[b])

    def load_wait(j, b):
        pltpu.make_async_copy(feat_hbm.at[src_b[b]], rows_v.at[b],
                              g_sems[b]).wait()
        pltpu.make_async_copy(coeff_hbm.at[wid, j], coeff_v.at[b],
                              c_sems[b]).wait()

    idx_load(0, 0)
    idx_load(1, 1)
    dst_load(0, 0)
    dst_load(1, 1)
    idx_wait(0, 0)
    load(0, 0)

    plsc.subcore_barrier()

    def pair(i0, carry):
        for b in range(NBUF):
            j = NBUF * i0 + b
            nb = 1 - b

            @pl.when(j < n_chunks)
            def _process():
                @pl.when(j + 1 < n_chunks)
                def _fire_next():
                    idx_wait(j + 1, nb)
                    load(j + 1, nb)

                load_wait(j, b)

                @pl.when(j >= 1)
                def _wait_prev_scatter():
                    pltpu.make_async_copy(
                        msg_v, acc_sh.at[dst_b[nb]], s_sem).wait()

                @pl.when(j + 1 < n_chunks)
                def _fire_dst():
                    dst_load(j + 1, nb)

                hi_mask = jnp.int32(-65536)

                def mrow(jr, carry2):
                    for k in range(D // (2 * LANES)):
                        slw = pl.ds(k * LANES, LANES)
                        wr = rows_v[b, jr, slw]
                        wc = coeff_v[b, jr, slw]
                        re = lax.bitcast_convert_type(wr << 16, jnp.float32)
                        ro = lax.bitcast_convert_type(wr & hi_mask, jnp.float32)
                        ce = lax.bitcast_convert_type(wc << 16, jnp.float32)
                        co = lax.bitcast_convert_type(wc & hi_mask, jnp.float32)
                        msg_v[jr, pl.ds(k * 2 * LANES, LANES)] = re * ce
                        msg_v[jr, pl.ds(k * 2 * LANES + LANES, LANES)] = ro * co
                    return carry2

                lax.fori_loop(0, C, mrow, 0)
                dst_wait(j, b)
                pltpu.async_copy(msg_v, acc_sh.at[dst_b[b]], s_sem,
                                 add=True)

                @pl.when(j + NBUF < n_chunks)
                def _fire_idx():
                    idx_load(j + NBUF, b)

        return carry

    lax.fori_loop(0, (n_chunks + NBUF - 1) // NBUF, pair, 0)

    pltpu.make_async_copy(msg_v, acc_sh.at[dst_b[(n_chunks - 1) % 2]],
                          s_sem).wait()

    plsc.subcore_barrier()
    pltpu.sync_copy(acc_sh.at[pl.ds(s * rpt, rpt), :],
                    out_hbm.at[c, pl.ds(s * rpt, rpt), :])


def _sc_call(features, coeff_g, src_g, dst_g, n_pad, chunk_e):
    N = features.shape[0]
    D = features.shape[1] * 2
    NW, n_chunks, C, _ = coeff_g.shape
    assert C == chunk_e and NW == NC * NS
    mesh = plsc.VectorSubcoreMesh(core_axis_name="c", subcore_axis_name="s")
    body = functools.partial(_sc_edge_body, n_pad, D, C, n_chunks)
    f = pl.kernel(
        body,
        out_type=jax.ShapeDtypeStruct((NC, n_pad, D), jnp.float32),
        mesh=mesh,
        compiler_params=pltpu.CompilerParams(use_tc_tiling_on_sc=False),
        scratch_types=[
            pltpu.VMEM((C,), jnp.int32),
            pltpu.VMEM((C,), jnp.int32),
            pltpu.VMEM((C,), jnp.int32),
            pltpu.VMEM((C,), jnp.int32),
            pltpu.VMEM((2, C, D // 2), jnp.int32),
            pltpu.VMEM((2, C, D // 2), jnp.int32),
            pltpu.VMEM((C, D), jnp.float32),
            pltpu.VMEM_SHARED((n_pad, D), jnp.float32),
            pltpu.SemaphoreType.DMA,
            pltpu.SemaphoreType.DMA,
            pltpu.SemaphoreType.DMA,
            pltpu.SemaphoreType.DMA,
            pltpu.SemaphoreType.DMA,
            pltpu.SemaphoreType.DMA,
            pltpu.SemaphoreType.DMA,
            pltpu.SemaphoreType.DMA,
            pltpu.SemaphoreType.DMA,
        ],
    )
    return f(features, coeff_g, src_g, dst_g)


def _gate_body(p_ref, o_ref):
    a = (p_ref[0] + p_ref[1]) * np.float32(1.0 / np.sqrt(AVG_DEG))
    o_ref[...] = a * jax.nn.sigmoid(a)


def _gate_call(partial, N, block_n):
    _, _, D = partial.shape
    grid = N // block_n
    return pl.pallas_call(
        _gate_body,
        grid=(grid,),
        in_specs=[pl.BlockSpec((NC, block_n, D), lambda i: (0, i, 0))],
        out_specs=pl.BlockSpec((block_n, D), lambda i: (i, 0)),
        out_shape=jax.ShapeDtypeStruct((N, D), jnp.float32),
    )(partial)


def kernel(features, edge_index, radii, W1, b1, W2, b2, W3, b3):
    N, D = features.shape
    E = radii.shape[0]
    H = W1.shape[1]
    NW = NC * NS

    perm = np.empty(D, dtype=np.int32)
    for blk in range(0, D, 2 * LANES):
        for i in range(LANES):
            perm[blk + 2 * i] = blk + i
            perm[blk + 2 * i + 1] = blk + LANES + i
    P = np.zeros((D, D), dtype=np.float32)
    P[perm, np.arange(D)] = 1.0

    feat_p = _prep_call(features, jnp.asarray(P), block_n=2000)
    feat_i = lax.bitcast_convert_type(
        feat_p.reshape(N, D // 2, 2), jnp.int32)

    block_e = 4000
    coeff = _coeff_call(radii.reshape(E // block_e, 1, block_e),
                        W1.T, b1.reshape(H, 1), W2.T, b2.reshape(H, 1),
                        W3[:, perm], b3[perm].reshape(1, D),
                        block_e=block_e)

    C = 80
    n_chunks = E // (NW * C)
    coeff_i = lax.bitcast_convert_type(
        coeff.reshape(E, D // 2, 2), jnp.int32)
    coeff_g = coeff_i.reshape(NW, n_chunks, C, D // 2)
    src_g = edge_index[0].reshape(NW, n_chunks, C)
    dst_g = edge_index[1].reshape(NW, n_chunks, C)

    n_pad = ((N + NS * C - 1) // (NS * C)) * (NS * C)
    partial = _sc_call(feat_i, coeff_g, src_g, dst_g, n_pad=n_pad,
                       chunk_e=C)
    return _gate_call(partial, N, block_n=2000)


# speedup vs baseline: 3.7068x; 3.7068x over previous
import functools

import jax
import jax.numpy as jnp
import numpy as np
from jax import lax
from jax.experimental import pallas as pl
from jax.experimental.pallas import tpu as pltpu
from jax.experimental.pallas import tpu_sc as plsc

MAX_RADIUS = 5.0
AVG_DEG = 32.0
NUM_BASIS = 16

NC = 2
NS = 16
LANES = 16

_COS_PI = (
    9.9999999989e-01, -4.9348021859e+00, 4.0587118172e+00, -1.3352602858e+00,
    2.3532082435e-01, -2.5785806878e-02, 1.9043274683e-03, -8.8690476959e-05,
)


def _coeff_body(r_ref, w1t_ref, b1_ref, w2t_ref, b2_ref, w3_ref, b3_ref,
                out_ref):
    r = r_ref[0]
    inv_step = np.float32((NUM_BASIS - 1) / MAX_RADIUS)
    k_col = lax.broadcasted_iota(
        jnp.int32, (NUM_BASIS, 1), 0).astype(jnp.float32)
    x = r * inv_step - k_col
    u = x * x
    p = jnp.full_like(u, np.float32(_COS_PI[-1]))
    for coef in _COS_PI[-2::-1]:
        p = p * u + np.float32(coef)
    basis = jnp.where(u < 1.0, 0.5 + 0.5 * p, 0.0)
    h = jnp.dot(w1t_ref[...], basis, preferred_element_type=jnp.float32)
    h = jnp.maximum(h + b1_ref[...], 0.0)
    h = jnp.dot(w2t_ref[...], h, preferred_element_type=jnp.float32)
    h = jnp.maximum(h + b2_ref[...], 0.0)
    out = lax.dot_general(h, w3_ref[...], (((0,), (0,)), ((), ())),
                          preferred_element_type=jnp.float32)
    out_ref[...] = out + b3_ref[...]


def _coeff_call(radii2, W1t, b1, W2t, b2, W3, b3, block_e):
    grid, _, BE = radii2.shape
    H, NB = W1t.shape
    D = W3.shape[1]
    assert BE == block_e
    full = lambda shape: pl.BlockSpec(shape, lambda i: (0, 0))
    return pl.pallas_call(
        _coeff_body,
        grid=(grid,),
        in_specs=[
            pl.BlockSpec((1, 1, block_e), lambda i: (i, 0, 0)),
            full((H, NB)),
            full((H, 1)),
            full((H, H)),
            full((H, 1)),
            full((H, D)),
            full((1, D)),
        ],
        out_specs=pl.BlockSpec((block_e, D), lambda i: (i, 0)),
        out_shape=jax.ShapeDtypeStruct((grid * block_e, D), jnp.float32),
    )(radii2, W1t, b1, W2t, b2, W3, b3)


def _sc_edge_body(N_pad, D, C, n_chunks,
                  feat_hbm, coeff_hbm, src_hbm, dst_hbm, out_hbm,
                  src_b0, src_b1, dst_b0, dst_b1, rows_v, coeff_v, msg_v,
                  acc_sh, g0, g1, c0, c1, is0, is1, id0, id1, s_sem):
    NBUF = 2
    src_b = (src_b0, src_b1)
    dst_b = (dst_b0, dst_b1)
    g_sems = (g0, g1)
    c_sems = (c0, c1)
    is_sems = (is0, is1)
    id_sems = (id0, id1)
    c = lax.axis_index("c")
    s = lax.axis_index("s")
    wid = s * NC + c
    rpt = N_pad // NS

    def zrow(j, carry):
        for k in range(D // LANES):
            msg_v[j, pl.ds(k * LANES, LANES)] = jnp.zeros((LANES,),
                                                          jnp.float32)
        return carry

    lax.fori_loop(0, C, zrow, 0)
    for t in range(rpt // C):
        pltpu.sync_copy(msg_v, acc_sh.at[pl.ds(s * rpt + t * C, C), :])

    def idx_load(j, b):
        pltpu.async_copy(src_hbm.at[wid, j], src_b[b], is_sems[b])

    def idx_wait(j, b):
        pltpu.make_async_copy(src_hbm.at[wid, j], src_b[b], is_sems[b]).wait()

    def dst_load(j, b):
        pltpu.async_copy(dst_hbm.at[wid, j], dst_b[b], id_sems[b])

    def dst_wait(j, b):
        pltpu.make_async_copy(dst_hbm.at[wid, j], dst_b[b], id_sems[b]).wait()

    def load(j, b):
        pltpu.async_copy(feat_hbm.at[src_b[b]], rows_v.at[b], g_sems[b])
        pltpu.async_copy(coeff_hbm.at[wid, j], coeff_v.at[b], c_sems[b])

    def load_wait(j, b):
        pltpu.make_async_copy(feat_hbm.at[src_b[b]], rows_v.at[b],
                              g_sems[b]).wait()
        pltpu.make_async_copy(coeff_hbm.at[wid, j], coeff_v.at[b],
                              c_sems[b]).wait()

    idx_load(0, 0)
    idx_load(1, 1)
    dst_load(0, 0)
    dst_load(1, 1)
    idx_wait(0, 0)
    load(0, 0)

    plsc.subcore_barrier()

    def pair(i0, carry):
        for b in range(NBUF):
            j = NBUF * i0 + b
            nb = 1 - b

            @pl.when(j < n_chunks)
            def _process():
                @pl.when(j + 1 < n_chunks)
                def _fire_next():
                    idx_wait(j + 1, nb)
                    load(j + 1, nb)

                load_wait(j, b)

                @pl.when(j >= 1)
                def _wait_prev_scatter():
                    pltpu.make_async_copy(
                        msg_v, acc_sh.at[dst_b[nb]], s_sem).wait()

                @pl.when(j + 1 < n_chunks)
                def _fire_dst():
                    dst_load(j + 1, nb)

                def mrow(jr, carry2):
                    for k in range(D // LANES):
                        sl = pl.ds(k * LANES, LANES)
                        msg_v[jr, sl] = rows_v[b, jr, sl] * coeff_v[b, jr, sl]
                    return carry2

                lax.fori_loop(0, C, mrow, 0)
                dst_wait(j, b)
                pltpu.async_copy(msg_v, acc_sh.at[dst_b[b]], s_sem,
                                 add=True)

                @pl.when(j + NBUF < n_chunks)
                def _fire_idx():
                    idx_load(j + NBUF, b)

        return carry

    lax.fori_loop(0, (n_chunks + NBUF - 1) // NBUF, pair, 0)

    pltpu.make_async_copy(msg_v, acc_sh.at[dst_b[(n_chunks - 1) % 2]],
                          s_sem).wait()

    plsc.subcore_barrier()
    pltpu.sync_copy(acc_sh.at[pl.ds(s * rpt, rpt), :],
                    out_hbm.at[c, pl.ds(s * rpt, rpt), :])


def _sc_call(features, coeff_g, src_g, dst_g, n_pad, chunk_e):
    N, D = features.shape
    NW, n_chunks, C, _ = coeff_g.shape
    assert C == chunk_e and NW == NC * NS
    mesh = plsc.VectorSubcoreMesh(core_axis_name="c", subcore_axis_name="s")
    body = functools.partial(_sc_edge_body, n_pad, D, C, n_chunks)
    f = pl.kernel(
        body,
        out_type=jax.ShapeDtypeStruct((NC, n_pad, D), jnp.float32),
        mesh=mesh,
        scratch_types=[
            pltpu.VMEM((C,), jnp.int32),
            pltpu.VMEM((C,), jnp.int32),
            pltpu.VMEM((C,), jnp.int32),
            pltpu.VMEM((C,), jnp.int32),
            pltpu.VMEM((2, C, D), jnp.float32),
            pltpu.VMEM((2, C, D), jnp.float32),
            pltpu.VMEM((C, D), jnp.float32),
            pltpu.VMEM_SHARED((n_pad, D), jnp.float32),
            pltpu.SemaphoreType.DMA,
            pltpu.SemaphoreType.DMA,
            pltpu.SemaphoreType.DMA,
            pltpu.SemaphoreType.DMA,
            pltpu.SemaphoreType.DMA,
            pltpu.SemaphoreType.DMA,
            pltpu.SemaphoreType.DMA,
            pltpu.SemaphoreType.DMA,
            pltpu.SemaphoreType.DMA,
        ],
    )
    return f(features, coeff_g, src_g, dst_g)


def _gate_body(p0_ref, p1_ref, o_ref):
    a = (p0_ref[0] + p0_ref[1] + p1_ref[0] + p1_ref[1]) * np.float32(
        1.0 / np.sqrt(AVG_DEG))
    o_ref[...] = a * jax.nn.sigmoid(a)


def _gate_call(partial0, partial1, N, block_n):
    _, _, D = partial0.shape
    grid = N // block_n
    spec = pl.BlockSpec((NC, block_n, D), lambda i: (0, i, 0))
    return pl.pallas_call(
        _gate_body,
        grid=(grid,),
        in_specs=[spec, spec],
        out_specs=pl.BlockSpec((block_n, D), lambda i: (i, 0)),
        out_shape=jax.ShapeDtypeStruct((N, D), jnp.float32),
    )(partial0, partial1)


def kernel(features, edge_index, radii, W1, b1, W2, b2, W3, b3):
    N, D = features.shape
    E = radii.shape[0]
    H = W1.shape[1]
    NW = NC * NS

    NSLICES = 2
    C = 40
    block_e = 4000
    Es = E // NSLICES
    n_chunks = Es // (NW * C)
    W1t, W2t = W1.T, W2.T
    b1c, b2c = b1.reshape(H, 1), b2.reshape(H, 1)
    b3r = b3.reshape(1, D)

    n_pad = ((N + NS * C - 1) // (NS * C)) * (NS * C)

    src = edge_index[0].reshape(NSLICES, NW, n_chunks, C)
    dst = edge_index[1].reshape(NSLICES, NW, n_chunks, C)
    radii_s = radii.reshape(NSLICES, Es // block_e, 1, block_e)

    partials = []
    for sl in range(NSLICES):
        coeff = _coeff_call(radii_s[sl], W1t, b1c, W2t, b2c, W3, b3r,
                            block_e=block_e)
        coeff_g = coeff.reshape(NW, n_chunks, C, D)
        partials.append(_sc_call(features, coeff_g, src[sl], dst[sl],
                                 n_pad=n_pad, chunk_e=C))

    return _gate_call(partials[0], partials[1], N, block_n=2000)
